# baseline (speedup 1.0000x reference)
import functools

import jax
import jax.numpy as jnp
from jax import lax
from jax.experimental import pallas as pl
from jax.experimental.pallas import tpu as pltpu
from jax.experimental.pallas import tpu_sc as plsc

NUM_CORES = 2
NUM_SUBCORES = 16
NUM_LANES = 16
L = 20
D = 64
IDX_CHUNK = 128

MAGIC20, SHIFT20 = 52429, 20
MAGIC26, SHIFT26 = 20165, 19


def _body(tok_hbm, lens_hbm, table_hbm, out_hbm, tok3_v, lens2_v, tok_v,
          zc_v, inv_v, v_v, o_v, t0_v, slab, sem, A, B, V):
    c = lax.axis_index("c")
    s = lax.axis_index("s")
    bags = (B // NUM_SUBCORES) * A
    b_per_s = B // NUM_SUBCORES
    n_tok = bags * L
    n_groups = bags // NUM_LANES
    n_chunks = n_tok // IDX_CHUNK
    d_per_c = D // NUM_CORES

    pltpu.sync_copy(lens_hbm.at[:, pl.ds(s * b_per_s, b_per_s)], lens2_v)

    lane = lax.broadcasted_iota(jnp.int32, (NUM_LANES,), 0)
    lane_l = lane * L

    n_blk = b_per_s // 16
    for blk in range(n_blk):
        pltpu.sync_copy(
            tok_hbm.at[:, :, pl.ds(s * b_per_s + blk * 16, 16)], tok3_v)

        def flat_tok(k, _):
            p = k * NUM_LANES + lane
            r = lax.shift_right_logical(p * MAGIC20, SHIFT20)
            j = p - r * L
            bl = lax.shift_right_logical(r * MAGIC26, SHIFT26)
            a = r - bl * A
            t = plsc.load_gather(tok3_v, [a, j, bl - blk * 16])
            tok_v[pl.ds(k * NUM_LANES, NUM_LANES)] = t
            return 0

        blk_chunks = 16 * A * L // NUM_LANES
        lax.fori_loop(blk * blk_chunks, (blk + 1) * blk_chunks, flat_tok, 0,
                      unroll=2)

    def stats(g, _):
        r = g * NUM_LANES + lane
        bl = lax.shift_right_logical(r * MAGIC26, SHIFT26)
        a = r - bl * A
        lens_i = plsc.load_gather(lens2_v, [a, bl])
        inv = 1.0 / jnp.maximum(lens_i.astype(jnp.float32), 1.0)
        z = jnp.zeros((NUM_LANES,), jnp.float32)
        for j in range(L):
            t = plsc.load_gather(tok_v, [g * (NUM_LANES * L) + lane_l + j])
            z = z + jnp.where(t == 0, 1.0, 0.0)
        zc_v[pl.ds(g * NUM_LANES, NUM_LANES)] = z
        inv_v[pl.ds(g * NUM_LANES, NUM_LANES)] = inv
        return 0

    lax.fori_loop(0, n_groups, stats, 0, unroll=2)

    def remap(k, _):
        t = tok_v[pl.ds(k * NUM_LANES, NUM_LANES)]
        tok_v[pl.ds(k * NUM_LANES, NUM_LANES)] = jnp.where(t == 0, 1, t)
        return 0

    lax.fori_loop(0, n_tok // NUM_LANES, remap, 0, unroll=2)

    zeros16 = jnp.full((NUM_LANES,), 0, jnp.int32)

    def dim_body(d, _):
        dg = c * d_per_c + d

        @pl.when(s == 0)
        def _():
            pltpu.sync_copy(table_hbm.at[pl.ds(dg, 1)], slab)
        plsc.subcore_barrier()

        pltpu.sync_copy(slab.at[pl.ds(0, 1), pl.ds(0, NUM_LANES)], t0_v)
        t0d = plsc.load_gather(t0_v, [zeros16, zeros16 + 1])

        for h in range(2):
            h_tok = h * (n_tok // 2)
            h_bag = h * (bags // 2)

            def fire(k, _):
                idx = tok_v.at[pl.ds(h_tok + k * IDX_CHUNK, IDX_CHUNK)]
                pltpu.async_copy(slab.at[0].at[idx],
                                 v_v.at[pl.ds(k * IDX_CHUNK, IDX_CHUNK)],
                                 sem)
                return 0

            lax.fori_loop(0, n_chunks // 2, fire, 0)
            pltpu.make_async_copy(
                slab.at[0].at[pl.ds(0, n_tok // 2)], v_v, sem).wait()

            def pool(g, _):
                base = g * (NUM_LANES * L)
                acc = plsc.load_gather(v_v, [base + lane_l])
                for j in range(1, L):
                    acc = acc + plsc.load_gather(v_v, [base + lane_l + j])
                gg = h_bag + g * NUM_LANES
                z = zc_v[pl.ds(gg, NUM_LANES)]
                inv = inv_v[pl.ds(gg, NUM_LANES)]
                o_v[0, pl.ds(gg, NUM_LANES)] = (acc - z * t0d) * inv
                return 0

            lax.fori_loop(0, n_groups // 2, pool, 0)
        pltpu.sync_copy(o_v, out_hbm.at[pl.ds(dg, 1), pl.ds(s * bags, bags)])
        plsc.subcore_barrier()
        return 0

    lax.fori_loop(0, d_per_c, dim_body, 0)


def kernel(token_ids, lengths, table):
    B, A, Ltok = token_ids.shape
    V, Dtab = table.shape
    assert Ltok == L and Dtab == D
    n_bags = B * A
    assert B % NUM_SUBCORES == 0 and D % NUM_CORES == 0
    assert (B // NUM_SUBCORES) * A * L % IDX_CHUNK == 0

    tok_t = jnp.transpose(token_ids.astype(jnp.int32), (1, 2, 0))
    lens_t = jnp.transpose(lengths.astype(jnp.int32), (1, 0))
    table_t = jnp.transpose(table, (1, 0))

    bags = (B // NUM_SUBCORES) * A
    mesh = plsc.VectorSubcoreMesh(core_axis_name="c", subcore_axis_name="s")
    body = functools.partial(_body, A=A, B=B, V=V)
    out_t = pl.kernel(
        body,
        out_type=jax.ShapeDtypeStruct((D, n_bags), jnp.float32),
        mesh=mesh,
        compiler_params=pltpu.CompilerParams(needs_layout_passes=False,
                                             use_tc_tiling_on_sc=False),
        scratch_types=[
            pltpu.VMEM((A, L, 16), jnp.int32),
            pltpu.VMEM((A, B // NUM_SUBCORES), jnp.int32),
            pltpu.VMEM((bags * L,), jnp.int32),
            pltpu.VMEM((bags,), jnp.float32),
            pltpu.VMEM((bags,), jnp.float32),
            pltpu.VMEM((bags * L // 2,), jnp.float32),
            pltpu.VMEM((1, bags), jnp.float32),
            pltpu.VMEM((1, NUM_LANES), jnp.float32),
            pltpu.VMEM_SHARED((1, V), jnp.float32),
            pltpu.SemaphoreType.DMA,
        ],
    )(tok_t, lens_t, table_t)
    return jnp.transpose(out_t, (1, 0)).reshape(B, A, D)


# speedup vs baseline: 8.0820x; 8.0820x over previous
import functools

import jax
import jax.numpy as jnp
from jax import lax
from jax.experimental import pallas as pl
from jax.experimental.pallas import tpu as pltpu
from jax.experimental.pallas import tpu_sc as plsc

NUM_CORES = 2
NUM_SUBCORES = 16
NUM_LANES = 16
NW = NUM_CORES * NUM_SUBCORES

L = 20
D = 64
NCB = D // NUM_LANES
GROUP = 32
IDX_CHUNK = 32
N_CHUNKS = GROUP * L // IDX_CHUNK

MAGIC20, SHIFT20 = 52429, 20
MAGIC26, SHIFT26 = 20165, 19


def _fire(table_hbm, tok_v, rows_buf, sem, g):
    g_tok = g * (GROUP * L)
    for q in range(N_CHUNKS):
        idx_ref = tok_v.at[pl.ds(g_tok + q * IDX_CHUNK, IDX_CHUNK)]
        dst = rows_buf.at[pl.ds(q * IDX_CHUNK, IDX_CHUNK)]
        pltpu.async_copy(table_hbm.at[idx_ref], dst, sem)


def _body(tok_hbm, lens_hbm, table_hbm, out_hbm, tok3_v, lens2_v, tok_v,
          lens_v, rows_a, rows_b, out_v, sc_v, t0_v, sem_a, sem_b,
          n_bags_per_w, n_groups, b_per_w, A):
    wid = lax.axis_index("s") * NUM_CORES + lax.axis_index("c")
    bag_base = wid * n_bags_per_w

    pltpu.sync_copy(tok_hbm.at[:, :, pl.ds(wid * b_per_w, b_per_w)], tok3_v)
    pltpu.sync_copy(lens_hbm.at[:, pl.ds(wid * b_per_w, b_per_w)], lens2_v)
    pltpu.sync_copy(table_hbm.at[pl.ds(0, 1)], t0_v)

    lane = lax.broadcasted_iota(jnp.int32, (NUM_LANES,), 0)
    lane_l = lane * L
    t0 = [t0_v[0, pl.ds(c * NUM_LANES, NUM_LANES)] for c in range(NCB)]

    def flat_tok(c, _):
        p = c * NUM_LANES + lane
        r = lax.shift_right_logical(p * MAGIC20, SHIFT20)
        j = p - r * L
        bl = lax.shift_right_logical(r * MAGIC26, SHIFT26)
        a = r - bl * A
        t = plsc.load_gather(tok3_v, [a, j, bl])
        tok_v[pl.ds(c * NUM_LANES, NUM_LANES)] = t
        return 0

    lax.fori_loop(0, n_bags_per_w * L // NUM_LANES, flat_tok, 0, unroll=2)

    def flat_len(c, _):
        r = c * NUM_LANES + lane
        bl = lax.shift_right_logical(r * MAGIC26, SHIFT26)
        a = r - bl * A
        v = plsc.load_gather(lens2_v, [a, bl])
        lens_v[pl.ds(c * NUM_LANES, NUM_LANES)] = v
        return 0

    lax.fori_loop(0, n_bags_per_w // NUM_LANES, flat_len, 0, unroll=2)

    def wait(rows_buf, sem):
        pltpu.make_async_copy(
            table_hbm.at[pl.ds(0, GROUP * L)], rows_buf, sem).wait()

    def compute(g, rows_buf):
        g_tok = g * (GROUP * L)
        for half in range(GROUP // NUM_LANES):
            lens_i = plsc.load_gather(
                lens_v, [g * GROUP + half * NUM_LANES + lane])
            inv = 1.0 / jnp.maximum(lens_i.astype(jnp.float32), 1.0)
            h_tok = g_tok + half * (NUM_LANES * L)
            z = jnp.zeros((NUM_LANES,), jnp.float32)
            for j in range(L):
                t = plsc.load_gather(tok_v, [h_tok + j + lane_l])
                z = z + jnp.where(t == 0, 1.0, 0.0)
            plsc.store_scatter(sc_v, [half * NUM_LANES + lane], z)
            plsc.store_scatter(sc_v, [GROUP + half * NUM_LANES + lane], inv)

        def bag_body(r, _):
            base = r * L
            accs = [rows_buf[base, pl.ds(c * NUM_LANES, NUM_LANES)]
                    for c in range(NCB)]
            for j in range(1, L):
                for c in range(NCB):
                    v = rows_buf[base + j, pl.ds(c * NUM_LANES, NUM_LANES)]
                    accs[c] = accs[c] + v
            rvec = jnp.full((NUM_LANES,), 0, jnp.int32) + r
            cv = plsc.load_gather(sc_v, [rvec])
            iv = plsc.load_gather(sc_v, [rvec + GROUP])
            for c in range(NCB):
                out_v[r, pl.ds(c * NUM_LANES, NUM_LANES)] = (
                    (accs[c] - cv * t0[c]) * iv)
            return 0

        lax.fori_loop(0, GROUP, bag_body, 0)
        pltpu.sync_copy(out_v, out_hbm.at[pl.ds(bag_base + g * GROUP, GROUP)])

    _fire(table_hbm, tok_v, rows_a, sem_a, 0)

    def pair_body(i, _):
        g = i * 2

        @pl.when(g + 1 < n_groups)
        def _():
            _fire(table_hbm, tok_v, rows_b, sem_b, g + 1)
        wait(rows_a, sem_a)
        compute(g, rows_a)

        @pl.when(g + 2 < n_groups)
        def _():
            _fire(table_hbm, tok_v, rows_a, sem_a, g + 2)

        @pl.when(g + 1 < n_groups)
        def _():
            wait(rows_b, sem_b)
            compute(g + 1, rows_b)
        return 0

    lax.fori_loop(0, (n_groups + 1) // 2, pair_body, 0)


def kernel(token_ids, lengths, table):
    B, A, Ltok = token_ids.shape
    assert Ltok == L and table.shape[1] == D
    n_bags = B * A
    assert n_bags % (NW * GROUP) == 0
    n_bags_per_w = n_bags // NW
    n_groups = n_bags_per_w // GROUP
    b_per_w = B // NW

    tok_t = jnp.transpose(token_ids.astype(jnp.int32), (1, 2, 0))
    lens_t = jnp.transpose(lengths.astype(jnp.int32), (1, 0))

    mesh = plsc.VectorSubcoreMesh(core_axis_name="c", subcore_axis_name="s")
    body = functools.partial(_body, n_bags_per_w=n_bags_per_w,
                             n_groups=n_groups, b_per_w=b_per_w, A=A)
    out = pl.kernel(
        body,
        out_type=jax.ShapeDtypeStruct((n_bags, D), jnp.float32),
        mesh=mesh,
        compiler_params=pltpu.CompilerParams(needs_layout_passes=False,
                                             use_tc_tiling_on_sc=False),
        scratch_types=[
            pltpu.VMEM((A, L, B // NW), jnp.int32),
            pltpu.VMEM((A, B // NW), jnp.int32),
            pltpu.VMEM((n_bags_per_w * L,), jnp.int32),
            pltpu.VMEM((n_bags_per_w,), jnp.int32),
            pltpu.VMEM((GROUP * L, D), jnp.float32),
            pltpu.VMEM((GROUP * L, D), jnp.float32),
            pltpu.VMEM((GROUP, D), jnp.float32),
            pltpu.VMEM((2 * GROUP,), jnp.float32),
            pltpu.VMEM((1, D), jnp.float32),
            pltpu.SemaphoreType.DMA,
            pltpu.SemaphoreType.DMA,
        ],
    )(tok_t, lens_t, table)
    return out.reshape(B, A, D)
